# baseline (speedup 1.0000x reference)
import math

import jax
import jax.numpy as jnp
from jax.experimental import pallas as pl
from jax.experimental.pallas import tpu as pltpu

K = 1000
S = 8
D = 1024
B = 4096
BETA = 5.0
ALPHA = 2.0

_XSCALE = BETA * math.log2(math.e)
_OSCALE = ALPHA * math.exp(-BETA)

_BB = 1024
_NBUF = 3


def _tda_kernel(x_ref, m_hbm, o_ref, xb_ref, ms_ref, sem):
    xb_ref[...] = (x_ref[...] * _XSCALE).astype(jnp.bfloat16)

    def copy(s):
        return pltpu.make_async_copy(
            m_hbm.at[:, pl.ds(s * D, D)], ms_ref.at[s % _NBUF], sem.at[s % _NBUF])

    for s in range(min(_NBUF - 1, S)):
        copy(s).start()

    acc = None
    for s in range(S):
        if s + _NBUF - 1 < S:
            copy(s + _NBUF - 1).start()
        copy(s).wait()
        mb = ms_ref[s % _NBUF].astype(jnp.bfloat16)
        a = jax.lax.dot_general(
            xb_ref[...], mb,
            dimension_numbers=(((1,), (1,)), ((), ())),
            preferred_element_type=jnp.float32,
        )
        e = jnp.exp2(a)
        acc = e if acc is None else acc + e
    o_ref[...] = acc * _OSCALE


def kernel(x, memory):
    m2 = memory.reshape(K, S * D)
    grid = (B // _BB,)
    return pl.pallas_call(
        _tda_kernel,
        grid=grid,
        in_specs=[
            pl.BlockSpec((_BB, D), lambda i: (i, 0)),
            pl.BlockSpec(memory_space=pltpu.MemorySpace.HBM),
        ],
        out_specs=pl.BlockSpec((_BB, K), lambda i: (i, 0)),
        out_shape=jax.ShapeDtypeStruct((B, K), jnp.float32),
        scratch_shapes=[
            pltpu.VMEM((_BB, D), jnp.bfloat16),
            pltpu.VMEM((_NBUF, K, D), jnp.float32),
            pltpu.SemaphoreType.DMA((_NBUF,)),
        ],
    )(x, m2)


# speedup vs baseline: 1.0409x; 1.0409x over previous
import math

import jax
import jax.numpy as jnp
from jax.experimental import pallas as pl
from jax.experimental.pallas import tpu as pltpu

K = 1000
S = 8
D = 1024
B = 4096
BETA = 5.0
ALPHA = 2.0

_XSCALE = BETA * math.log2(math.e)
_OSCALE = ALPHA * math.exp(-BETA)

_BB = 2048
_KSPLIT = (0, 512, K)
_NBUF = 3
_KMAX = 512


def _tda_kernel(x_ref, m_hbm, o_ref, xb_ref, ms_ref, sem):
    xb_ref[...] = (x_ref[...] * _XSCALE).astype(jnp.bfloat16)

    def tile(t):
        kh, s = divmod(t, S)
        k0, k1 = _KSPLIT[kh], _KSPLIT[kh + 1]
        return k0, k1 - k0, s

    def copy(t):
        k0, kw, s = tile(t)
        return pltpu.make_async_copy(
            m_hbm.at[pl.ds(k0, kw), pl.ds(s * D, D)],
            ms_ref.at[t % _NBUF, pl.ds(0, kw)],
            sem.at[t % _NBUF])

    ntiles = (len(_KSPLIT) - 1) * S
    for t in range(_NBUF - 1):
        copy(t).start()

    for kh in range(len(_KSPLIT) - 1):
        k0, k1 = _KSPLIT[kh], _KSPLIT[kh + 1]
        kw = k1 - k0
        acc = None
        for s in range(S):
            t = kh * S + s
            if t + _NBUF - 1 < ntiles:
                copy(t + _NBUF - 1).start()
            copy(t).wait()
            mb = ms_ref[t % _NBUF, pl.ds(0, kw)].astype(jnp.bfloat16)
            a = jax.lax.dot_general(
                xb_ref[...], mb,
                dimension_numbers=(((1,), (1,)), ((), ())),
                preferred_element_type=jnp.float32,
            )
            e = jnp.exp2(a)
            acc = e if acc is None else acc + e
        o_ref[:, k0:k1] = acc * _OSCALE


def kernel(x, memory):
    m2 = memory.reshape(K, S * D)
    grid = (B // _BB,)
    return pl.pallas_call(
        _tda_kernel,
        grid=grid,
        in_specs=[
            pl.BlockSpec((_BB, D), lambda i: (i, 0)),
            pl.BlockSpec(memory_space=pltpu.MemorySpace.HBM),
        ],
        out_specs=pl.BlockSpec((_BB, K), lambda i: (i, 0)),
        out_shape=jax.ShapeDtypeStruct((B, K), jnp.float32),
        scratch_shapes=[
            pltpu.VMEM((_BB, D), jnp.bfloat16),
            pltpu.VMEM((_NBUF, _KMAX, D), jnp.float32),
            pltpu.SemaphoreType.DMA((_NBUF,)),
        ],
    )(x, m2)
